# baseline (speedup 1.0000x reference)
import functools

import jax
import jax.numpy as jnp
from jax import lax
from jax.experimental import pallas as pl
from jax.experimental.pallas import tpu as pltpu
from jax.experimental.pallas import tpu_sc as plsc

B, D, T = 32, 128, 4096
K = 1024
TBLK = 512
M = B * T


def _codes_kernel(lat_ref, cb_ref, codes_ref):
    lat = lat_ref[0]
    cb = cb_ref[...]

    xwT = jax.lax.dot_general(
        cb, lat, (((1,), (0,)), ((), ())),
        preferred_element_type=jnp.float32)

    x2 = jnp.sum(lat * lat, axis=0, keepdims=True)
    w2 = jnp.sum(cb * cb, axis=1, keepdims=True)
    d2T = (x2 - 2.0 * xwT) + w2

    m = jnp.min(d2T, axis=0, keepdims=True)
    ids = jax.lax.broadcasted_iota(jnp.int32, d2T.shape, 0)
    cand = jnp.where(d2T == m, ids, K)
    codes_ref[0, 0, :] = jnp.min(cand, axis=0)


NSEG = 4
BSEG = B // NSEG


def _compute_codes_seg(latents, code_embed, seg):
    b0 = seg * BSEG
    codes3 = pl.pallas_call(
        _codes_kernel,
        grid=(BSEG, T // TBLK),
        in_specs=[
            pl.BlockSpec((1, D, TBLK), lambda b, t: (b + b0, 0, t)),
            pl.BlockSpec((K, D), lambda b, t: (0, 0)),
        ],
        out_specs=pl.BlockSpec((1, 1, TBLK), lambda b, t: (b, 0, t)),
        out_shape=jax.ShapeDtypeStruct((BSEG, 1, T), jnp.int32),
    )(latents, code_embed)
    return codes3.reshape(BSEG, T)


CH = 128
NBUF = 4


MSEG = M // NSEG


def _make_sc_gather():
    info = plsc.get_sparse_core_info()
    NC, NS = info.num_cores, info.num_subcores
    NW = NC * NS
    w_rows = MSEG // NW
    n_chunks = w_rows // CH
    mesh = plsc.VectorSubcoreMesh(core_axis_name="c", subcore_axis_name="s")

    @functools.partial(
        pl.kernel, mesh=mesh,
        out_type=jax.ShapeDtypeStruct((M, D), jnp.float32),
        scratch_types=[
            pltpu.VMEM((n_chunks, CH), jnp.int32),
            pltpu.VMEM((NBUF, CH, D), jnp.float32),
            pltpu.SemaphoreType.DMA,
            pltpu.SemaphoreType.DMA,
        ],
    )
    def gather_k(i0, i1, i2, i3, table_hbm, out_hbm,
                 idx_v, rows_v, sem_g, sem_s):
        wid = lax.axis_index("s") * NC + lax.axis_index("c")
        for q, idx_hbm in enumerate((i0, i1, i2, i3)):
            base = q * MSEG + wid * w_rows
            pltpu.sync_copy(idx_hbm.at[pl.ds(wid * n_chunks, n_chunks)], idx_v)

            def body(g, carry):
                gath = [
                    pltpu.async_copy(
                        table_hbm.at[idx_v.at[g * NBUF + b]],
                        rows_v.at[b], sem_g)
                    for b in range(NBUF)
                ]
                stores = []
                for b in range(NBUF):
                    gath[b].wait()
                    stores.append(pltpu.async_copy(
                        rows_v.at[b],
                        out_hbm.at[pl.ds(base + (g * NBUF + b) * CH, CH)],
                        sem_s))
                for s in stores:
                    s.wait()
                return carry

            lax.fori_loop(0, n_chunks // NBUF, body, 0)

    return gather_k


_sc_gather = _make_sc_gather()


def kernel(latents, code_embed):
    codes_segs = []
    idx_segs = []
    for seg in range(NSEG):
        c = _compute_codes_seg(latents, code_embed, seg)
        codes_segs.append(c)
        idx_segs.append(c.reshape(MSEG // CH, CH))
    quant = _sc_gather(*idx_segs, code_embed)
    codes = jnp.concatenate(codes_segs, axis=0)
    return quant.reshape(B, T, D), codes


# speedup vs baseline: 1.2232x; 1.2232x over previous
import functools

import jax
import jax.numpy as jnp
from jax import lax
from jax.experimental import pallas as pl
from jax.experimental.pallas import tpu as pltpu
from jax.experimental.pallas import tpu_sc as plsc

B, D, T = 32, 128, 4096
K = 1024
TBLK = 512
M = B * T


def _codes_kernel(lat_ref, cb_ref, codes_ref):
    lat = lat_ref[0]
    cb = cb_ref[...]

    lat2 = lat * (-2.0)
    xwT = jax.lax.dot_general(
        cb, lat2, (((1,), (0,)), ((), ())),
        preferred_element_type=jnp.float32)
    w2 = jnp.sum(cb * cb, axis=1, keepdims=True)
    v = xwT + w2

    ids = jax.lax.broadcasted_iota(jnp.int32, v.shape, 0)
    r = K
    while r > 8:
        h = r // 2
        a, b_ = v[:h], v[h:]
        keep = a <= b_
        v = jnp.where(keep, a, b_)
        ids = jnp.where(keep, ids[:h], ids[h:])
        r = h
    m = jnp.min(v, axis=0, keepdims=True)
    cand = jnp.where(v == m, ids, K)
    codes_ref[0, 0, :] = jnp.min(cand, axis=0)


NSEG = 4
BSEG = B // NSEG


def _compute_codes_seg(latents, code_embed, seg):
    b0 = seg * BSEG
    codes3 = pl.pallas_call(
        _codes_kernel,
        grid=(BSEG, T // TBLK),
        in_specs=[
            pl.BlockSpec((1, D, TBLK), lambda b, t: (b + b0, 0, t)),
            pl.BlockSpec((K, D), lambda b, t: (0, 0)),
        ],
        out_specs=pl.BlockSpec((1, 1, TBLK), lambda b, t: (b, 0, t)),
        out_shape=jax.ShapeDtypeStruct((BSEG, 1, T), jnp.int32),
    )(latents, code_embed)
    return codes3.reshape(BSEG, T)


CH = 128
NBUF = 4


MSEG = M // NSEG


def _make_sc_gather(n_rows):
    info = plsc.get_sparse_core_info()
    NC, NS = info.num_cores, info.num_subcores
    NW = NC * NS
    b_per_w = n_rows // NW
    n_chunks = b_per_w // CH
    mesh = plsc.VectorSubcoreMesh(core_axis_name="c", subcore_axis_name="s")

    @functools.partial(
        pl.kernel, mesh=mesh,
        out_type=jax.ShapeDtypeStruct((n_rows, D), jnp.float32),
        scratch_types=[
            pltpu.VMEM((n_chunks, CH), jnp.int32),
            pltpu.VMEM((NBUF, CH, D), jnp.float32),
            pltpu.SemaphoreType.DMA,
            pltpu.SemaphoreType.DMA,
        ],
    )
    def gather_k(idx_hbm, table_hbm, out_hbm, idx_v, rows_v, sem_g, sem_s):
        wid = lax.axis_index("s") * NC + lax.axis_index("c")
        base = wid * b_per_w
        pltpu.sync_copy(idx_hbm.at[pl.ds(wid * n_chunks, n_chunks)], idx_v)

        def body(g, carry):
            gath = [
                pltpu.async_copy(
                    table_hbm.at[idx_v.at[g * NBUF + b]], rows_v.at[b], sem_g)
                for b in range(NBUF)
            ]
            stores = []
            for b in range(NBUF):
                gath[b].wait()
                stores.append(pltpu.async_copy(
                    rows_v.at[b],
                    out_hbm.at[pl.ds(base + (g * NBUF + b) * CH, CH)],
                    sem_s))
            for s in stores:
                s.wait()
            return carry

        lax.fori_loop(0, n_chunks // NBUF, body, 0)

    return gather_k


_sc_gather = _make_sc_gather(MSEG)


def kernel(latents, code_embed):
    codes_segs = []
    quant_segs = []
    for seg in range(NSEG):
        c = _compute_codes_seg(latents, code_embed, seg)
        codes_segs.append(c)
        quant_segs.append(_sc_gather(c.reshape(MSEG // CH, CH), code_embed))
    codes = jnp.concatenate(codes_segs, axis=0)
    quant = jnp.concatenate(quant_segs, axis=0)
    return quant.reshape(B, T, D), codes


# speedup vs baseline: 1.3488x; 1.1027x over previous
import functools

import jax
import jax.numpy as jnp
from jax import lax
from jax.experimental import pallas as pl
from jax.experimental.pallas import tpu as pltpu
from jax.experimental.pallas import tpu_sc as plsc

B, D, T = 32, 128, 4096
K = 1024
TBLK = 512
M = B * T


def _codes_kernel(lat_ref, cb_ref, codes_ref):
    lat = lat_ref[0]
    cb = cb_ref[...]

    lat2 = lat * (-2.0)
    xwT = jax.lax.dot_general(
        cb, lat2, (((1,), (0,)), ((), ())),
        preferred_element_type=jnp.float32)
    w2 = jnp.sum(cb * cb, axis=1, keepdims=True)
    v = xwT + w2

    ids = jax.lax.broadcasted_iota(jnp.int32, v.shape, 0)
    r = K
    while r > 8:
        h = r // 2
        a, b_ = v[:h], v[h:]
        keep = a <= b_
        v = jnp.where(keep, a, b_)
        ids = jnp.where(keep, ids[:h], ids[h:])
        r = h
    m = jnp.min(v, axis=0, keepdims=True)
    cand = jnp.where(v == m, ids, K)
    codes_ref[0, 0, :] = jnp.min(cand, axis=0)


def _codes_quant_kernel(lat_ref, cb_ref, codes_ref, quant_ref):
    lat = lat_ref[0]
    cb = cb_ref[...]
    lat2 = lat * (-2.0)
    xwT = jax.lax.dot_general(
        cb, lat2, (((1,), (0,)), ((), ())),
        preferred_element_type=jnp.float32)
    w2 = jnp.sum(cb * cb, axis=1, keepdims=True)
    v = xwT + w2
    full_ids = jax.lax.broadcasted_iota(jnp.int32, v.shape, 0)
    ids = full_ids
    r = K
    while r > 8:
        h = r // 2
        a, b_ = v[:h], v[h:]
        keep = a <= b_
        v = jnp.where(keep, a, b_)
        ids = jnp.where(keep, ids[:h], ids[h:])
        r = h
    m = jnp.min(v, axis=0, keepdims=True)
    cand = jnp.where(v == m, ids, K)
    code = jnp.min(cand, axis=0, keepdims=True)
    codes_ref[0, 0, :] = code[0]
    ohT = (full_ids == code).astype(jnp.float32)
    quant_ref[0] = jax.lax.dot_general(
        ohT, cb, (((0,), (0,)), ((), ())),
        preferred_element_type=jnp.float32)


NSEG = 4
BSEG = B // NSEG


def _compute_codes_seg(latents, code_embed, seg):
    b0 = seg * BSEG
    codes3 = pl.pallas_call(
        _codes_kernel,
        grid=(BSEG, T // TBLK),
        in_specs=[
            pl.BlockSpec((1, D, TBLK), lambda b, t: (b + b0, 0, t)),
            pl.BlockSpec((K, D), lambda b, t: (0, 0)),
        ],
        out_specs=pl.BlockSpec((1, 1, TBLK), lambda b, t: (b, 0, t)),
        out_shape=jax.ShapeDtypeStruct((BSEG, 1, T), jnp.int32),
    )(latents, code_embed)
    return codes3.reshape(BSEG, T)


def _compute_codes_quant_seg(latents, code_embed, seg):
    b0 = seg * BSEG
    codes3, quant = pl.pallas_call(
        _codes_quant_kernel,
        grid=(BSEG, T // TBLK),
        in_specs=[
            pl.BlockSpec((1, D, TBLK), lambda b, t: (b + b0, 0, t)),
            pl.BlockSpec((K, D), lambda b, t: (0, 0)),
        ],
        out_specs=[
            pl.BlockSpec((1, 1, TBLK), lambda b, t: (b, 0, t)),
            pl.BlockSpec((1, TBLK, D), lambda b, t: (b, t, 0)),
        ],
        out_shape=[
            jax.ShapeDtypeStruct((BSEG, 1, T), jnp.int32),
            jax.ShapeDtypeStruct((BSEG, T, D), jnp.float32),
        ],
    )(latents, code_embed)
    return codes3.reshape(BSEG, T), quant.reshape(MSEG, D)


CH = 128
NBUF = 4


MSEG = M // NSEG


def _make_sc_gather(n_rows):
    info = plsc.get_sparse_core_info()
    NC, NS = info.num_cores, info.num_subcores
    NW = NC * NS
    b_per_w = n_rows // NW
    n_chunks = b_per_w // CH
    mesh = plsc.VectorSubcoreMesh(core_axis_name="c", subcore_axis_name="s")

    @functools.partial(
        pl.kernel, mesh=mesh,
        out_type=jax.ShapeDtypeStruct((n_rows, D), jnp.float32),
        scratch_types=[
            pltpu.VMEM((n_chunks, CH), jnp.int32),
            pltpu.VMEM((NBUF, CH, D), jnp.float32),
            pltpu.SemaphoreType.DMA,
            pltpu.SemaphoreType.DMA,
        ],
    )
    def gather_k(idx_hbm, table_hbm, out_hbm, idx_v, rows_v, sem_g, sem_s):
        wid = lax.axis_index("s") * NC + lax.axis_index("c")
        base = wid * b_per_w
        pltpu.sync_copy(idx_hbm.at[pl.ds(wid * n_chunks, n_chunks)], idx_v)

        def body(g, carry):
            gath = [
                pltpu.async_copy(
                    table_hbm.at[idx_v.at[g * NBUF + b]], rows_v.at[b], sem_g)
                for b in range(NBUF)
            ]
            stores = []
            for b in range(NBUF):
                gath[b].wait()
                stores.append(pltpu.async_copy(
                    rows_v.at[b],
                    out_hbm.at[pl.ds(base + (g * NBUF + b) * CH, CH)],
                    sem_s))
            for s in stores:
                s.wait()
            return carry

        lax.fori_loop(0, n_chunks // NBUF, body, 0)

    return gather_k


_sc_gather = _make_sc_gather(MSEG)


def kernel(latents, code_embed):
    codes_segs = []
    quant_segs = []
    for seg in range(NSEG - 1):
        c = _compute_codes_seg(latents, code_embed, seg)
        codes_segs.append(c)
        quant_segs.append(_sc_gather(c.reshape(MSEG // CH, CH), code_embed))
    c_last, q_last = _compute_codes_quant_seg(latents, code_embed, NSEG - 1)
    codes_segs.append(c_last)
    quant_segs.append(q_last)
    codes = jnp.concatenate(codes_segs, axis=0)
    quant = jnp.concatenate(quant_segs, axis=0)
    return quant.reshape(B, T, D), codes
